# baseline (speedup 1.0000x reference)
import functools

import jax
import jax.numpy as jnp
from jax import lax
from jax.experimental import pallas as pl
from jax.experimental.pallas import tpu as pltpu
from jax.experimental.pallas import tpu_sc as plsc

_L = 16


@functools.lru_cache(maxsize=None)
def _build(B, V, D, nc, ns):
    nw = nc * ns
    b_per_w = B // nw
    n_groups = b_per_w // _L
    nfull = V // 128
    mesh = plsc.VectorSubcoreMesh(core_axis_name="c", subcore_axis_name="s")

    @functools.partial(
        pl.kernel,
        mesh=mesh,
        out_type=jax.ShapeDtypeStruct((D, B), jnp.float32),
        scratch_types=[
            pltpu.VMEM((b_per_w,), jnp.int32),
            pltpu.VMEM((_L, D, 128), jnp.float32),
            pltpu.VMEM((D, b_per_w), jnp.float32),
            [pltpu.SemaphoreType.DMA for _ in range(_L)],
        ],
        compiler_params=pltpu.CompilerParams(
            use_tc_tiling_on_sc=True, needs_layout_passes=False
        ),
    )
    def gather_kernel(idx_hbm, table_hbm, tail_hbm, out_hbm, idx_v, blk_v, out_v, sems):
        wid = lax.axis_index("s") * nc + lax.axis_index("c")
        pltpu.sync_copy(idx_hbm.at[wid], idx_v)
        iota = lax.iota(jnp.int32, _L)

        def fetch(slot, i):
            blkid = i >> 7

            @pl.when(blkid != nfull)
            def _():
                off = pl.multiple_of(blkid * 128, 128)
                pltpu.async_copy(
                    table_hbm.at[:, pl.ds(off, 128)], blk_v.at[slot], sems[slot]
                )

            @pl.when(blkid == nfull)
            def _():
                pltpu.async_copy(tail_hbm, blk_v.at[slot], sems[slot])

        def extract(slot, i, r):
            pltpu.make_async_copy(
                table_hbm.at[:, pl.ds(0, 128)], blk_v.at[slot], sems[slot]
            ).wait()
            word = jnp.full((_L,), i & 127, jnp.int32)
            vals = plsc.load_gather(blk_v.at[slot], [iota, word])
            plsc.store_scatter(out_v, [iota, jnp.full((_L,), r, jnp.int32)], vals)

        def group(g, carry):
            it0 = g * _L
            iv = idx_v[pl.ds(it0, _L)]
            for r in range(_L):
                fetch(r, iv[r])
            for r in range(_L):
                extract(r, iv[r], it0 + r)
            return carry

        lax.fori_loop(0, n_groups, group, 0)

        base = pl.multiple_of(wid * b_per_w, 128)
        pltpu.sync_copy(out_v, out_hbm.at[:, pl.ds(base, b_per_w)])

    return gather_kernel


def kernel(idx, context_hat):
    B = idx.shape[0]
    V, D = context_hat.shape
    info = plsc.get_sparse_core_info()
    nc, ns = info.num_cores, info.num_subcores
    nw = nc * ns
    idx_2d = idx.reshape(B).astype(jnp.int32).reshape(nw, B // nw)
    rem = V % 128
    tail = jnp.zeros((D, 128), jnp.float32).at[:, :rem].set(context_hat[V - rem :].T)
    out_t = _build(B, V, D, nc, ns)(idx_2d, context_hat.T, tail)
    return out_t.T
